# Initial kernel scaffold; baseline (speedup 1.0000x reference)
import jax
import jax.numpy as jnp
from jax.experimental import pallas as pl


def kernel(x, edge_index, W1, b1, W2, b2):
    raise NotImplementedError("write your pallas kernel here")


# speedup vs baseline: 5.2871x; 5.2871x over previous
import functools

import jax
import jax.numpy as jnp
from jax import lax
from jax.experimental import pallas as pl
from jax.experimental.pallas import tpu as pltpu
from jax.experimental.pallas import tpu_sc as plsc

N_NODES = 10000
N_EDGES = 320000
D_IN = 128
D_HID = 128
N_CLASSES = 40
D2P = 48

NC = 2
NS = 16
NW = NC * NS
NP = 10240
EPW = N_EDGES // NW
K = 80
NCH = EPW // K
RPS = NP // NS
ZR = 128
NCOPY = RPS // ZR

_mesh = lambda: plsc.VectorSubcoreMesh(core_axis_name="c", subcore_axis_name="s")


def _zero_fill(zbuf, rows, d):
    zv = jnp.zeros((16,), jnp.float32)

    def body(i, carry):
        for j in range(d // 16):
            zbuf[i, pl.ds(j * 16, 16)] = zv
        return carry

    lax.fori_loop(0, rows, body, 0)


def _make_deg():
    @functools.partial(
        pl.kernel,
        mesh=_mesh(),
        compiler_params=pltpu.CompilerParams(use_tc_tiling_on_sc=False),
        out_type=jax.ShapeDtypeStruct((NC, NP, 16), jnp.float32),
        scratch_types=[
            pltpu.VMEM_SHARED((NP, 16), jnp.float32),
            pltpu.VMEM((ZR, 16), jnp.float32),
            pltpu.VMEM((K, 16), jnp.float32),
            pltpu.VMEM((K,), jnp.int32),
        ],
    )
    def degk(dst_hbm, out_hbm, acc, zbuf, ones, didx):
        c = lax.axis_index("c")
        s = lax.axis_index("s")
        wid = s * NC + c

        _zero_fill(zbuf, ZR, 16)
        ov = jnp.ones((16,), jnp.float32)

        def fill_ones(i, carry):
            ones[i, pl.ds(0, 16)] = ov
            return carry

        lax.fori_loop(0, K, fill_ones, 0)
        for r in range(NCOPY):
            pltpu.sync_copy(zbuf, acc.at[pl.ds(s * RPS + r * ZR, ZR)])
        plsc.subcore_barrier()

        base = wid * EPW

        def body(j, carry):
            pltpu.sync_copy(dst_hbm.at[pl.ds(base + j * K, K)], didx)
            pltpu.sync_copy(ones, acc.at[didx], add=True)
            return carry

        lax.fori_loop(0, NCH, body, 0)
        plsc.subcore_barrier()
        pltpu.sync_copy(acc.at[pl.ds(s * RPS, RPS)],
                        out_hbm.at[c, pl.ds(s * RPS, RPS)])

    return degk


def _make_agg(d):
    @functools.partial(
        pl.kernel,
        mesh=_mesh(),
        compiler_params=pltpu.CompilerParams(use_tc_tiling_on_sc=(d % 128 == 0)),
        out_type=jax.ShapeDtypeStruct((NC, NP, d), jnp.float32),
        scratch_types=[
            pltpu.VMEM_SHARED((NP, d), jnp.float32),
            pltpu.VMEM((ZR, d), jnp.float32),
            pltpu.VMEM((K,), jnp.int32),
            pltpu.VMEM((K,), jnp.int32),
            pltpu.VMEM((K, d), jnp.float32),
            pltpu.SemaphoreType.DMA,
        ],
    )
    def aggk(table_hbm, src_hbm, dst_hbm, out_hbm, acc, zbuf, sidx, didx,
             rows, sem):
        c = lax.axis_index("c")
        s = lax.axis_index("s")
        wid = s * NC + c

        _zero_fill(zbuf, ZR, d)
        for r in range(NCOPY):
            pltpu.sync_copy(zbuf, acc.at[pl.ds(s * RPS + r * ZR, ZR)])
        plsc.subcore_barrier()

        base = wid * EPW

        def body(j, carry):
            off = base + j * K
            pltpu.sync_copy(src_hbm.at[pl.ds(off, K)], sidx)
            pltpu.sync_copy(dst_hbm.at[pl.ds(off, K)], didx)
            pltpu.async_copy(table_hbm.at[sidx], rows, sem).wait()
            pltpu.sync_copy(rows, acc.at[didx], add=True)
            return carry

        lax.fori_loop(0, NCH, body, 0)
        plsc.subcore_barrier()
        pltpu.sync_copy(acc.at[pl.ds(s * RPS, RPS)],
                        out_hbm.at[c, pl.ds(s * RPS, RPS)])

    return aggk


_deg_call = _make_deg()
_agg128 = _make_agg(D_HID)
_agg48 = _make_agg(D2P)

BN = 1000


def _b_body(d_ref, x_ref, w_ref, t_ref, n_ref):
    dg = d_ref[0, :, 0:1] + d_ref[1, :, 0:1]
    nrm = lax.rsqrt(jnp.maximum(dg, 1.0))
    t_ref[...] = jnp.dot(x_ref[...], w_ref[...],
                         preferred_element_type=jnp.float32) * nrm
    n_ref[...] = nrm


def _tc_b(deg2, x, W1):
    return pl.pallas_call(
        _b_body,
        grid=(N_NODES // BN,),
        in_specs=[
            pl.BlockSpec((NC, BN, 16), lambda i: (0, i, 0)),
            pl.BlockSpec((BN, D_IN), lambda i: (i, 0)),
            pl.BlockSpec((D_IN, D_HID), lambda i: (0, 0)),
        ],
        out_specs=[
            pl.BlockSpec((BN, D_HID), lambda i: (i, 0)),
            pl.BlockSpec((BN, 1), lambda i: (i, 0)),
        ],
        out_shape=[
            jax.ShapeDtypeStruct((N_NODES, D_HID), jnp.float32),
            jax.ShapeDtypeStruct((N_NODES, 1), jnp.float32),
        ],
    )(deg2, x, W1)


def _d_body(p_ref, n_ref, b1_ref, w2_ref, t2_ref):
    agg = p_ref[0] + p_ref[1]
    nrm = n_ref[...]
    h = jnp.maximum(agg * nrm + b1_ref[...], 0.0)
    t2_ref[...] = jnp.dot(h, w2_ref[...],
                          preferred_element_type=jnp.float32) * nrm


def _tc_d(p1, norm, b1r, W2p):
    return pl.pallas_call(
        _d_body,
        grid=(N_NODES // BN,),
        in_specs=[
            pl.BlockSpec((NC, BN, D_HID), lambda i: (0, i, 0)),
            pl.BlockSpec((BN, 1), lambda i: (i, 0)),
            pl.BlockSpec((1, D_HID), lambda i: (0, 0)),
            pl.BlockSpec((D_HID, D2P), lambda i: (0, 0)),
        ],
        out_specs=pl.BlockSpec((BN, D2P), lambda i: (i, 0)),
        out_shape=jax.ShapeDtypeStruct((N_NODES, D2P), jnp.float32),
    )(p1, norm, b1r, W2p)


def _f_body(q_ref, n_ref, b2_ref, o_ref):
    agg = q_ref[0] + q_ref[1]
    o_ref[...] = (agg * n_ref[...])[:, :N_CLASSES] + b2_ref[...]


def _tc_f(p2, norm, b2r):
    return pl.pallas_call(
        _f_body,
        grid=(N_NODES // BN,),
        in_specs=[
            pl.BlockSpec((NC, BN, D2P), lambda i: (0, i, 0)),
            pl.BlockSpec((BN, 1), lambda i: (i, 0)),
            pl.BlockSpec((1, N_CLASSES), lambda i: (0, 0)),
        ],
        out_specs=pl.BlockSpec((BN, N_CLASSES), lambda i: (i, 0)),
        out_shape=jax.ShapeDtypeStruct((N_NODES, N_CLASSES), jnp.float32),
    )(p2, norm, b2r)


def kernel(x, edge_index, W1, b1, W2, b2):
    src = edge_index[0]
    dst = edge_index[1]

    degp = _deg_call(dst)
    table1, norm = _tc_b(degp, x, W1)
    p1 = _agg128(table1, src, dst)
    W2p = jnp.pad(W2, ((0, 0), (0, D2P - N_CLASSES)))
    table2 = _tc_d(p1, norm, b1.reshape(1, D_HID), W2p)
    p2 = _agg48(table2, src, dst)
    out = _tc_f(p2, norm, b2.reshape(1, N_CLASSES))
    return out


# speedup vs baseline: 7.0480x; 1.3331x over previous
import functools

import jax
import jax.numpy as jnp
from jax import lax
from jax.experimental import pallas as pl
from jax.experimental.pallas import tpu as pltpu
from jax.experimental.pallas import tpu_sc as plsc

N_NODES = 10000
N_EDGES = 320000
D_IN = 128
D_HID = 128
N_CLASSES = 40
D2P = 64

NC = 2
NS = 16
NW = NC * NS
NP = 10240
K = 80
NCH = N_EDGES // NW // K
NCHS = N_EDGES // NS // K
RPS = NP // NS
ZR = 128
NCOPY = RPS // ZR

_mesh = lambda: plsc.VectorSubcoreMesh(core_axis_name="c", subcore_axis_name="s")


def _zero_fill(zbuf, rows, d):
    zv = jnp.zeros((16,), jnp.float32)

    def body(i, carry):
        for j in range(d // 16):
            zbuf[i, pl.ds(j * 16, 16)] = zv
        return carry

    lax.fori_loop(0, rows, body, 0)


def _make_deg():
    @functools.partial(
        pl.kernel,
        mesh=_mesh(),
        compiler_params=pltpu.CompilerParams(use_tc_tiling_on_sc=False),
        out_type=jax.ShapeDtypeStruct((NC, NP, 16), jnp.float32),
        scratch_types=[
            pltpu.VMEM_SHARED((NP, 16), jnp.float32),
            pltpu.VMEM((ZR, 16), jnp.float32),
            pltpu.VMEM((K, 16), jnp.float32),
            pltpu.VMEM((NCH, K), jnp.int32),
            pltpu.SemaphoreType.DMA,
        ],
    )
    def degk(dst_hbm, out_hbm, acc, zbuf, ones, dstall, sem):
        c = lax.axis_index("c")
        s = lax.axis_index("s")
        wid = s * NC + c

        _zero_fill(zbuf, ZR, 16)
        ov = jnp.ones((16,), jnp.float32)

        def fill_ones(i, carry):
            ones[i, pl.ds(0, 16)] = ov
            return carry

        lax.fori_loop(0, K, fill_ones, 0)
        pltpu.sync_copy(dst_hbm.at[wid], dstall)
        for r in range(NCOPY):
            pltpu.sync_copy(zbuf, acc.at[pl.ds(s * RPS + r * ZR, ZR)])
        plsc.subcore_barrier()

        def body(g, carry):
            for t in range(5):
                pltpu.async_copy(ones, acc.at[dstall.at[5 * g + t]], sem,
                                 add=True)
            for t in range(5):
                pltpu.make_async_copy(ones, acc.at[dstall.at[5 * g + t]],
                                      sem).wait()
            return carry

        lax.fori_loop(0, NCH // 5, body, 0)
        plsc.subcore_barrier()
        pltpu.sync_copy(acc.at[pl.ds(s * RPS, RPS)],
                        out_hbm.at[c, pl.ds(s * RPS, RPS)])

    return degk


def _make_agg(d):
    @functools.partial(
        pl.kernel,
        mesh=_mesh(),
        compiler_params=pltpu.CompilerParams(use_tc_tiling_on_sc=False),
        out_type=jax.ShapeDtypeStruct((NC, NP, d), jnp.float32),
        scratch_types=[
            pltpu.VMEM_SHARED((NP, d), jnp.float32),
            pltpu.VMEM((ZR, d), jnp.float32),
            pltpu.VMEM((NCHS, K), jnp.int32),
            pltpu.VMEM((NCHS, K), jnp.int32),
            pltpu.VMEM((K, d), jnp.float32),
            pltpu.VMEM((K, d), jnp.float32),
            pltpu.SemaphoreType.DMA,
            pltpu.SemaphoreType.DMA,
        ],
    )
    def aggk(table_hbm, src_hbm, dst_hbm, out_hbm, acc, zbuf, srcall, dstall,
             rows0, rows1, sem0, sem1):
        c = lax.axis_index("c")
        s = lax.axis_index("s")

        _zero_fill(zbuf, ZR, d)
        pltpu.sync_copy(src_hbm.at[s], srcall)
        pltpu.sync_copy(dst_hbm.at[s], dstall)

        offv = jnp.full((16,), c * N_NODES, jnp.int32)

        def addoff(i, carry):
            for t in range(K // 16):
                sl = pl.ds(16 * t, 16)
                srcall[i, sl] = srcall[i, sl] + offv
            return carry

        lax.fori_loop(0, NCHS, addoff, 0)
        for r in range(NCOPY):
            pltpu.sync_copy(zbuf, acc.at[pl.ds(s * RPS + r * ZR, ZR)])
        plsc.subcore_barrier()

        pltpu.async_copy(table_hbm.at[srcall.at[0]], rows0, sem0)

        def body(i, carry):
            j = 2 * i
            pltpu.make_async_copy(table_hbm.at[srcall.at[j]], rows0,
                                  sem0).wait()
            pltpu.async_copy(table_hbm.at[srcall.at[j + 1]], rows1, sem1)
            pltpu.sync_copy(rows0, acc.at[dstall.at[j]], add=True)
            pltpu.make_async_copy(table_hbm.at[srcall.at[j + 1]], rows1,
                                  sem1).wait()
            pltpu.async_copy(table_hbm.at[srcall.at[j + 2]], rows0, sem0)
            pltpu.sync_copy(rows1, acc.at[dstall.at[j + 1]], add=True)
            return carry

        lax.fori_loop(0, (NCHS - 2) // 2, body, 0)
        pltpu.make_async_copy(table_hbm.at[srcall.at[NCHS - 2]], rows0,
                              sem0).wait()
        pltpu.async_copy(table_hbm.at[srcall.at[NCHS - 1]], rows1, sem1)
        pltpu.sync_copy(rows0, acc.at[dstall.at[NCHS - 2]], add=True)
        pltpu.make_async_copy(table_hbm.at[srcall.at[NCHS - 1]], rows1,
                              sem1).wait()
        pltpu.sync_copy(rows1, acc.at[dstall.at[NCHS - 1]], add=True)

        plsc.subcore_barrier()
        pltpu.sync_copy(acc.at[pl.ds(s * RPS, RPS)],
                        out_hbm.at[c, pl.ds(s * RPS, RPS)])

    return aggk


_deg_call = _make_deg()
_agg64 = _make_agg(D_HID // 2)
_agg32 = _make_agg(D2P // 2)

BN = 1000
H1 = D_HID // 2
H2 = D2P // 2


def _b_body(d_ref, x_ref, w_ref, t_ref, n_ref):
    dg = d_ref[0, :, 0:1] + d_ref[1, :, 0:1]
    nrm = lax.rsqrt(jnp.maximum(dg, 1.0))
    t1 = jnp.dot(x_ref[...], w_ref[...],
                 preferred_element_type=jnp.float32) * nrm
    t_ref[0] = t1[:, :H1]
    t_ref[1] = t1[:, H1:]
    n_ref[...] = nrm


def _tc_b(degp, x, W1):
    return pl.pallas_call(
        _b_body,
        grid=(N_NODES // BN,),
        in_specs=[
            pl.BlockSpec((NC, BN, 16), lambda i: (0, i, 0)),
            pl.BlockSpec((BN, D_IN), lambda i: (i, 0)),
            pl.BlockSpec((D_IN, D_HID), lambda i: (0, 0)),
        ],
        out_specs=[
            pl.BlockSpec((NC, BN, H1), lambda i: (0, i, 0)),
            pl.BlockSpec((BN, 1), lambda i: (i, 0)),
        ],
        out_shape=[
            jax.ShapeDtypeStruct((NC, N_NODES, H1), jnp.float32),
            jax.ShapeDtypeStruct((N_NODES, 1), jnp.float32),
        ],
    )(degp, x, W1)


def _d_body(p_ref, n_ref, b1_ref, w2_ref, t2_ref):
    agg = jnp.concatenate([p_ref[0], p_ref[1]], axis=1)
    nrm = n_ref[...]
    h = jnp.maximum(agg * nrm + b1_ref[...], 0.0)
    t2 = jnp.dot(h, w2_ref[...], preferred_element_type=jnp.float32) * nrm
    t2_ref[0] = t2[:, :H2]
    t2_ref[1] = t2[:, H2:]


def _tc_d(p1, norm, b1r, W2p):
    return pl.pallas_call(
        _d_body,
        grid=(N_NODES // BN,),
        in_specs=[
            pl.BlockSpec((NC, BN, H1), lambda i: (0, i, 0)),
            pl.BlockSpec((BN, 1), lambda i: (i, 0)),
            pl.BlockSpec((1, D_HID), lambda i: (0, 0)),
            pl.BlockSpec((D_HID, D2P), lambda i: (0, 0)),
        ],
        out_specs=pl.BlockSpec((NC, BN, H2), lambda i: (0, i, 0)),
        out_shape=jax.ShapeDtypeStruct((NC, N_NODES, H2), jnp.float32),
    )(p1, norm, b1r, W2p)


def _f_body(q_ref, n_ref, b2_ref, o_ref):
    agg = jnp.concatenate([q_ref[0], q_ref[1]], axis=1)
    o_ref[...] = (agg * n_ref[...])[:, :N_CLASSES] + b2_ref[...]


def _tc_f(p2, norm, b2r):
    return pl.pallas_call(
        _f_body,
        grid=(N_NODES // BN,),
        in_specs=[
            pl.BlockSpec((NC, BN, H2), lambda i: (0, i, 0)),
            pl.BlockSpec((BN, 1), lambda i: (i, 0)),
            pl.BlockSpec((1, N_CLASSES), lambda i: (0, 0)),
        ],
        out_specs=pl.BlockSpec((BN, N_CLASSES), lambda i: (i, 0)),
        out_shape=jax.ShapeDtypeStruct((N_NODES, N_CLASSES), jnp.float32),
    )(p2, norm, b2r)


def kernel(x, edge_index, W1, b1, W2, b2):
    src_s = edge_index[0].reshape(NS, NCHS, K)
    dst_s = edge_index[1].reshape(NS, NCHS, K)
    dst_w = edge_index[1].reshape(NW, NCH, K)

    degp = _deg_call(dst_w)
    t1, norm = _tc_b(degp, x, W1)
    p1 = _agg64(t1.reshape(NC * N_NODES, H1), src_s, dst_s)
    W2p = jnp.pad(W2, ((0, 0), (0, D2P - N_CLASSES)))
    t2 = _tc_d(p1, norm, b1.reshape(1, D_HID), W2p)
    p2 = _agg32(t2.reshape(NC * N_NODES, H2), src_s, dst_s)
    out = _tc_f(p2, norm, b2.reshape(1, N_CLASSES))
    return out


# speedup vs baseline: 10.3808x; 1.4729x over previous
import functools

import jax
import jax.numpy as jnp
from jax import lax
from jax.experimental import pallas as pl
from jax.experimental.pallas import tpu as pltpu
from jax.experimental.pallas import tpu_sc as plsc

N_NODES = 10000
N_EDGES = 320000
D_IN = 128
D_HID = 128
N_CLASSES = 40
D2P = 64

NC = 2
NS = 16
NW = NC * NS
NP = 10240
K = 80
NCH = N_EDGES // NW // K
NCHS = N_EDGES // NS // K
RPS = NP // NS
ZR = 128
NCOPY = RPS // ZR

_mesh = lambda: plsc.VectorSubcoreMesh(core_axis_name="c", subcore_axis_name="s")


def _zero_fill(zbuf, rows, d):
    zv = jnp.zeros((16,), jnp.float32)

    def body(i, carry):
        for j in range(d // 16):
            zbuf[i, pl.ds(j * 16, 16)] = zv
        return carry

    lax.fori_loop(0, rows, body, 0)


def _make_deg():
    @functools.partial(
        pl.kernel,
        mesh=_mesh(),
        compiler_params=pltpu.CompilerParams(use_tc_tiling_on_sc=False),
        out_type=jax.ShapeDtypeStruct((NC, NP, 16), jnp.float32),
        scratch_types=[
            pltpu.VMEM_SHARED((NP, 16), jnp.float32),
            pltpu.VMEM((ZR, 16), jnp.float32),
            pltpu.VMEM((K, 16), jnp.float32),
            pltpu.VMEM((NCH, K), jnp.int32),
            pltpu.SemaphoreType.DMA,
        ],
    )
    def degk(dst_hbm, out_hbm, acc, zbuf, ones, dstall, sem):
        c = lax.axis_index("c")
        s = lax.axis_index("s")
        wid = s * NC + c

        _zero_fill(zbuf, ZR, 16)
        ov = jnp.ones((16,), jnp.float32)

        def fill_ones(i, carry):
            ones[i, pl.ds(0, 16)] = ov
            return carry

        lax.fori_loop(0, K, fill_ones, 0)
        pltpu.sync_copy(dst_hbm.at[wid], dstall)
        for r in range(NCOPY):
            pltpu.sync_copy(zbuf, acc.at[pl.ds(s * RPS + r * ZR, ZR)])
        plsc.subcore_barrier()

        def body(j, carry):
            pltpu.sync_copy(ones, acc.at[dstall.at[j]], add=True)
            return carry

        lax.fori_loop(0, NCH, body, 0)
        plsc.subcore_barrier()
        pltpu.sync_copy(acc.at[pl.ds(s * RPS, RPS)],
                        out_hbm.at[c, pl.ds(s * RPS, RPS)])

    return degk


def _make_agg(d):
    @functools.partial(
        pl.kernel,
        mesh=_mesh(),
        compiler_params=pltpu.CompilerParams(use_tc_tiling_on_sc=False),
        out_type=jax.ShapeDtypeStruct((NC, NP, d), jnp.float32),
        scratch_types=[
            pltpu.VMEM_SHARED((NP, d), jnp.float32),
            pltpu.VMEM((ZR, d), jnp.float32),
            pltpu.VMEM((NCHS, K), jnp.int32),
            pltpu.VMEM((NCHS, K), jnp.int32),
            pltpu.VMEM((K, d), jnp.float32),
            pltpu.VMEM((K, d), jnp.float32),
            pltpu.VMEM((K, d), jnp.float32),
            pltpu.VMEM((K, d), jnp.float32),
            pltpu.SemaphoreType.DMA,
            pltpu.SemaphoreType.DMA,
            pltpu.SemaphoreType.DMA,
            pltpu.SemaphoreType.DMA,
            pltpu.SemaphoreType.DMA,
            pltpu.SemaphoreType.DMA,
            pltpu.SemaphoreType.DMA,
            pltpu.SemaphoreType.DMA,
        ],
    )
    def aggk(table_hbm, src_hbm, dst_hbm, out_hbm, acc, zbuf, srcall, dstall,
             rows0, rows1, rows2, rows3,
             semg0, semg1, semg2, semg3, sems0, sems1, sems2, sems3):
        c = lax.axis_index("c")
        s = lax.axis_index("s")
        rows = (rows0, rows1, rows2, rows3)
        semg = (semg0, semg1, semg2, semg3)
        sems = (sems0, sems1, sems2, sems3)

        _zero_fill(zbuf, ZR, d)
        pltpu.sync_copy(src_hbm.at[s], srcall)
        pltpu.sync_copy(dst_hbm.at[s], dstall)

        offv = jnp.full((16,), c * N_NODES, jnp.int32)

        def addoff(i, carry):
            for t in range(K // 16):
                sl = pl.ds(16 * t, 16)
                srcall[i, sl] = srcall[i, sl] + offv
            return carry

        lax.fori_loop(0, NCHS, addoff, 0)
        for r in range(NCOPY):
            pltpu.sync_copy(zbuf, acc.at[pl.ds(s * RPS + r * ZR, ZR)])
        plsc.subcore_barrier()

        def step(j, b, wait_prev_scatter, issue_next_gather):
            b1 = (b + 3) % 4
            b2 = (b + 2) % 4
            pltpu.make_async_copy(table_hbm.at[srcall.at[j]], rows[b],
                                  semg[b]).wait()
            if wait_prev_scatter:
                pltpu.make_async_copy(rows[b1], acc.at[dstall.at[j - 1]],
                                      sems[b1]).wait()
            pltpu.async_copy(rows[b], acc.at[dstall.at[j]], sems[b], add=True)
            if issue_next_gather:
                pltpu.async_copy(table_hbm.at[srcall.at[j + 2]], rows[b2],
                                 semg[b2])

        pltpu.async_copy(table_hbm.at[srcall.at[0]], rows0, semg0)
        pltpu.async_copy(table_hbm.at[srcall.at[1]], rows1, semg1)
        step(0, 0, False, True)
        step(1, 1, True, True)
        step(2, 2, True, True)
        step(3, 3, True, True)

        def body(g, carry):
            j0 = 4 * g
            for t in range(4):
                step(j0 + t, t, True, True)
            return carry

        lax.fori_loop(1, (NCHS - 2) // 4, body, 0)
        step(NCHS - 2, 0, True, False)
        step(NCHS - 1, 1, True, False)
        pltpu.make_async_copy(rows1, acc.at[dstall.at[NCHS - 1]],
                              sems1).wait()

        plsc.subcore_barrier()
        pltpu.sync_copy(acc.at[pl.ds(s * RPS, RPS)],
                        out_hbm.at[c, pl.ds(s * RPS, RPS)])

    return aggk


_deg_call = _make_deg()
_agg64 = _make_agg(D_HID // 2)
_agg32 = _make_agg(D2P // 2)

BN = 1000
H1 = D_HID // 2
H2 = D2P // 2


def _b_body(d_ref, x_ref, w_ref, t_ref, n_ref):
    dg = d_ref[0, :, 0:1] + d_ref[1, :, 0:1]
    nrm = lax.rsqrt(jnp.maximum(dg, 1.0))
    t1 = jnp.dot(x_ref[...], w_ref[...],
                 preferred_element_type=jnp.float32) * nrm
    t_ref[0] = t1[:, :H1]
    t_ref[1] = t1[:, H1:]
    n_ref[...] = nrm


def _tc_b(degp, x, W1):
    return pl.pallas_call(
        _b_body,
        grid=(N_NODES // BN,),
        in_specs=[
            pl.BlockSpec((NC, BN, 16), lambda i: (0, i, 0)),
            pl.BlockSpec((BN, D_IN), lambda i: (i, 0)),
            pl.BlockSpec((D_IN, D_HID), lambda i: (0, 0)),
        ],
        out_specs=[
            pl.BlockSpec((NC, BN, H1), lambda i: (0, i, 0)),
            pl.BlockSpec((BN, 1), lambda i: (i, 0)),
        ],
        out_shape=[
            jax.ShapeDtypeStruct((NC, N_NODES, H1), jnp.float32),
            jax.ShapeDtypeStruct((N_NODES, 1), jnp.float32),
        ],
    )(degp, x, W1)


def _d_body(p_ref, n_ref, b1_ref, w2_ref, t2_ref):
    agg = jnp.concatenate([p_ref[0], p_ref[1]], axis=1)
    nrm = n_ref[...]
    h = jnp.maximum(agg * nrm + b1_ref[...], 0.0)
    t2 = jnp.dot(h, w2_ref[...], preferred_element_type=jnp.float32) * nrm
    t2_ref[0] = t2[:, :H2]
    t2_ref[1] = t2[:, H2:]


def _tc_d(p1, norm, b1r, W2p):
    return pl.pallas_call(
        _d_body,
        grid=(N_NODES // BN,),
        in_specs=[
            pl.BlockSpec((NC, BN, H1), lambda i: (0, i, 0)),
            pl.BlockSpec((BN, 1), lambda i: (i, 0)),
            pl.BlockSpec((1, D_HID), lambda i: (0, 0)),
            pl.BlockSpec((D_HID, D2P), lambda i: (0, 0)),
        ],
        out_specs=pl.BlockSpec((NC, BN, H2), lambda i: (0, i, 0)),
        out_shape=jax.ShapeDtypeStruct((NC, N_NODES, H2), jnp.float32),
    )(p1, norm, b1r, W2p)


def _f_body(q_ref, n_ref, b2_ref, o_ref):
    agg = jnp.concatenate([q_ref[0], q_ref[1]], axis=1)
    o_ref[...] = (agg * n_ref[...])[:, :N_CLASSES] + b2_ref[...]


def _tc_f(p2, norm, b2r):
    return pl.pallas_call(
        _f_body,
        grid=(N_NODES // BN,),
        in_specs=[
            pl.BlockSpec((NC, BN, H2), lambda i: (0, i, 0)),
            pl.BlockSpec((BN, 1), lambda i: (i, 0)),
            pl.BlockSpec((1, N_CLASSES), lambda i: (0, 0)),
        ],
        out_specs=pl.BlockSpec((BN, N_CLASSES), lambda i: (i, 0)),
        out_shape=jax.ShapeDtypeStruct((N_NODES, N_CLASSES), jnp.float32),
    )(p2, norm, b2r)


def kernel(x, edge_index, W1, b1, W2, b2):
    src_s = edge_index[0].reshape(NS, NCHS, K)
    dst_s = edge_index[1].reshape(NS, NCHS, K)
    dst_w = edge_index[1].reshape(NW, NCH, K)

    degp = _deg_call(dst_w)
    t1, norm = _tc_b(degp, x, W1)
    p1 = _agg64(t1.reshape(NC * N_NODES, H1), src_s, dst_s)
    W2p = jnp.pad(W2, ((0, 0), (0, D2P - N_CLASSES)))
    t2 = _tc_d(p1, norm, b1.reshape(1, D_HID), W2p)
    p2 = _agg32(t2.reshape(NC * N_NODES, H2), src_s, dst_s)
    out = _tc_f(p2, norm, b2.reshape(1, N_CLASSES))
    return out
